# Initial kernel scaffold; baseline (speedup 1.0000x reference)
import jax
import jax.numpy as jnp
from jax.experimental import pallas as pl


def kernel(x, pos_table):
    raise NotImplementedError("write your pallas kernel here")


# speedup vs baseline: 2.6425x; 2.6425x over previous
import functools

import jax
import jax.numpy as jnp
from jax import lax
from jax.experimental import pallas as pl
from jax.experimental.pallas import tpu as pltpu
from jax.experimental.pallas import tpu_sc as plsc


def kernel(x, pos_table):
    B, S, D = x.shape
    dtype = pos_table.dtype

    info = plsc.get_sparse_core_info()
    NC, NS = info.num_cores, info.num_subcores
    NW = NC * NS
    rows_per_w = S // NW
    CHUNK = 16
    nchunks = rows_per_w // CHUNK

    mesh = plsc.VectorSubcoreMesh(core_axis_name="c", subcore_axis_name="s")

    @functools.partial(
        pl.kernel,
        mesh=mesh,
        out_type=jax.ShapeDtypeStruct((B, S, D), dtype),
        scratch_types=[
            pltpu.VMEM((CHUNK, D), dtype),
        ],
    )
    def broadcast_rows(table_hbm, out_hbm, buf):
        wid = lax.axis_index("s") * NC + lax.axis_index("c")
        row0 = wid * rows_per_w
        for c in range(nchunks):
            base = row0 + c * CHUNK
            pltpu.sync_copy(table_hbm.at[pl.ds(base, CHUNK)], buf)
            for b in range(B):
                pltpu.sync_copy(buf, out_hbm.at[b, pl.ds(base, CHUNK)])

    return broadcast_rows(pos_table)


# speedup vs baseline: 3.0444x; 1.1521x over previous
import functools

import jax
import jax.numpy as jnp
from jax import lax
from jax.experimental import pallas as pl
from jax.experimental.pallas import tpu as pltpu
from jax.experimental.pallas import tpu_sc as plsc


def kernel(x, pos_table):
    B, S, D = x.shape
    dtype = pos_table.dtype

    info = plsc.get_sparse_core_info()
    NC, NS = info.num_cores, info.num_subcores
    NW = NC * NS
    rows_per_w = S // NW
    CHUNK = 32
    nchunks = rows_per_w // CHUNK

    mesh = plsc.VectorSubcoreMesh(core_axis_name="c", subcore_axis_name="s")

    @functools.partial(
        pl.kernel,
        mesh=mesh,
        out_type=jax.ShapeDtypeStruct((B, S, D), dtype),
        scratch_types=[
            pltpu.VMEM((2, CHUNK, D), dtype),
            pltpu.SemaphoreType.DMA,
            pltpu.SemaphoreType.DMA,
        ],
    )
    def broadcast_rows(table_hbm, out_hbm, buf, lsem, ssem):
        wid = lax.axis_index("s") * NC + lax.axis_index("c")
        row0 = wid * rows_per_w

        loads = [None] * nchunks
        stores = [None] * nchunks

        def start_load(c):
            loads[c] = pltpu.async_copy(
                table_hbm.at[pl.ds(row0 + c * CHUNK, CHUNK)], buf.at[c % 2], lsem
            )

        start_load(0)
        for c in range(nchunks):
            if c >= 1:
                for h in stores[c - 1]:
                    h.wait()
            if c + 1 < nchunks:
                start_load(c + 1)
            loads[c].wait()
            stores[c] = [
                pltpu.async_copy(
                    buf.at[c % 2], out_hbm.at[b, pl.ds(row0 + c * CHUNK, CHUNK)], ssem
                )
                for b in range(B)
            ]
        for h in stores[nchunks - 1]:
            h.wait()

    return broadcast_rows(pos_table)


# speedup vs baseline: 3.0770x; 1.0107x over previous
import functools

import jax
import jax.numpy as jnp
from jax import lax
from jax.experimental import pallas as pl
from jax.experimental.pallas import tpu as pltpu
from jax.experimental.pallas import tpu_sc as plsc


def kernel(x, pos_table):
    B, S, D = x.shape
    dtype = pos_table.dtype

    info = plsc.get_sparse_core_info()
    NC, NS = info.num_cores, info.num_subcores
    NW = NC * NS
    rows_per_w = S // NW
    CHUNK = 32
    nchunks = rows_per_w // CHUNK

    mesh = plsc.VectorSubcoreMesh(core_axis_name="c", subcore_axis_name="s")

    NBUF = 3

    @functools.partial(
        pl.kernel,
        mesh=mesh,
        out_type=jax.ShapeDtypeStruct((B, S, D), dtype),
        scratch_types=[
            pltpu.VMEM((NBUF, CHUNK, D), dtype),
            pltpu.SemaphoreType.DMA,
            pltpu.SemaphoreType.DMA,
        ],
    )
    def broadcast_rows(table_hbm, out_hbm, buf, lsem, ssem):
        wid = lax.axis_index("s") * NC + lax.axis_index("c")
        row0 = wid * rows_per_w

        loads = [None] * nchunks
        stores = [None] * nchunks

        def start_load(c):
            loads[c] = pltpu.async_copy(
                table_hbm.at[pl.ds(row0 + c * CHUNK, CHUNK)], buf.at[c % NBUF], lsem
            )

        for n in range(min(NBUF, nchunks)):
            start_load(n)
        for c in range(nchunks):
            if c >= 1:
                for h in stores[c - 1]:
                    h.wait()
                n = (c - 1) + NBUF
                if n < nchunks:
                    start_load(n)
            loads[c].wait()
            stores[c] = [
                pltpu.async_copy(
                    buf.at[c % NBUF], out_hbm.at[b, pl.ds(row0 + c * CHUNK, CHUNK)], ssem
                )
                for b in range(B)
            ]
        for h in stores[nchunks - 1]:
            h.wait()

    return broadcast_rows(pos_table)
